# baseline (speedup 1.0000x reference)
import functools

import jax
import jax.numpy as jnp
import numpy as np
from jax import lax
from jax.experimental import pallas as pl
from jax.experimental.pallas import tpu as pltpu
from jax.experimental.pallas import tpu_sc as plsc

_SVD_DIM = 64
_SVD_ITER = 5
_SVD_SEEDS = (0, 1)

_NC = 2
_NS = 16
_NW = _NC * _NS


def _tsvd(A, k, n_iter, seed):
    key = jax.random.key(seed)
    n = A.shape[1]
    Omega = jax.random.normal(key, (n, k + 10), dtype=A.dtype)
    Y = A @ Omega
    for _ in range(n_iter):
        Y = A @ (A.T @ Y)
    Q, _ = jnp.linalg.qr(Y)
    Bm = Q.T @ A
    Ub, s, Vt = jnp.linalg.svd(Bm, full_matrices=False)
    U = Q @ Ub
    return U[:, :k] * s[:k]


def _sc_gather(table, idx3):
    NW, NB, BT = idx3.shape
    N, D = table.shape
    M = NW * NB * BT
    mesh = plsc.VectorSubcoreMesh(core_axis_name="c", subcore_axis_name="s")

    @functools.partial(
        pl.kernel,
        out_type=jax.ShapeDtypeStruct((M, D), jnp.float32),
        mesh=mesh,
        scratch_types=[
            pltpu.VMEM((NB, BT), jnp.int32),
            pltpu.VMEM((BT, D), jnp.float32),
            pltpu.SemaphoreType.DMA,
        ],
    )
    def k(table_hbm, idx_hbm, out_hbm, idx_v, rows_v, sem):
        c = lax.axis_index("c")
        s = lax.axis_index("s")
        wid = s * _NC + c
        base = wid * (NB * BT)
        pltpu.sync_copy(idx_hbm.at[wid], idx_v)

        def body(j, carry):
            pltpu.async_copy(table_hbm.at[idx_v.at[j]], rows_v, sem).wait()
            pltpu.sync_copy(rows_v, out_hbm.at[pl.ds(base + j * BT, BT)])
            return carry

        lax.fori_loop(0, NB, body, 0)

    return k(table, idx3)


def _sc_scatter(vals, src3, dst3, zeros):
    NW, NB, BT = src3.shape
    E, D = vals.shape
    N = zeros.shape[0]
    RPW = N // _NS
    mesh = plsc.VectorSubcoreMesh(core_axis_name="c", subcore_axis_name="s")

    @functools.partial(
        pl.kernel,
        out_type=jax.ShapeDtypeStruct((_NC, N, D), jnp.float32),
        mesh=mesh,
        scratch_types=[
            pltpu.VMEM((NB, BT), jnp.int32),
            pltpu.VMEM((NB, BT), jnp.int32),
            pltpu.VMEM((BT, D), jnp.float32),
            pltpu.VMEM_SHARED((N, D), jnp.float32),
        ],
    )
    def k(vals_hbm, src_hbm, dst_hbm, zeros_hbm, out_hbm, idxs_v, idxd_v,
          rows_v, acc):
        c = lax.axis_index("c")
        s = lax.axis_index("s")
        wid = s * _NC + c
        pltpu.sync_copy(zeros_hbm.at[pl.ds(s * RPW, RPW)],
                        acc.at[pl.ds(s * RPW, RPW)])
        pltpu.sync_copy(src_hbm.at[wid], idxs_v)
        pltpu.sync_copy(dst_hbm.at[wid], idxd_v)
        plsc.subcore_barrier()
        base = wid * (NB * BT)

        def body(j, carry):
            pltpu.sync_copy(vals_hbm.at[pl.ds(base + j * BT, BT)], rows_v)
            pltpu.sync_copy(rows_v, acc.at[idxs_v.at[j]], add=True)
            pltpu.sync_copy(rows_v, acc.at[idxd_v.at[j]], add=True)
            return carry

        lax.fori_loop(0, NB, body, 0)
        plsc.subcore_barrier()
        pltpu.sync_copy(acc.at[pl.ds(s * RPW, RPW)],
                        out_hbm.at[c, pl.ds(s * RPW, RPW)])

    return k(vals, src3, dst3, zeros)


def _mlp_body(gs_ref, gd_ref, w1_ref, b1_ref, w2_ref, b2_ref, h1e_ref,
              u_ref, ersum_ref):
    gs = gs_ref[...]
    gd = gd_ref[...]
    w1a = w1_ref[0:64, :]
    w1b = w1_ref[64:128, :]
    b1 = b1_ref[...]
    w2 = w2_ref[...]
    b2 = b2_ref[...]
    h1e = h1e_ref[...]

    def seedpart(cs):
        h = jnp.maximum(
            jnp.dot(gs[:, cs:cs + 64], w1a, preferred_element_type=jnp.float32)
            + jnp.dot(gd[:, cs:cs + 64], w1b, preferred_element_type=jnp.float32)
            + b1, 0.0)
        return jnp.dot(h, w2, preferred_element_type=jnp.float32) + b2

    er0 = seedpart(0)
    er1 = seedpart(64)
    u0 = jnp.dot(er0, h1e, preferred_element_type=jnp.float32)
    u1 = jnp.dot(er1, h1e, preferred_element_type=jnp.float32)
    u_ref[...] = jnp.concatenate([u0, u1], axis=1)

    @pl.when(pl.program_id(0) == 0)
    def _():
        ersum_ref[...] = jnp.zeros_like(ersum_ref)

    ersum_ref[...] += jnp.sum(er0 + er1, axis=0, keepdims=True)


def _tc_mlp(G, W1, b1, W2, b2, H1e, E):
    BLK = 2048
    nblk = E // BLK
    return pl.pallas_call(
        _mlp_body,
        grid=(nblk,),
        in_specs=[
            pl.BlockSpec((BLK, 128), lambda i: (i, 0)),
            pl.BlockSpec((BLK, 128), lambda i, nblk=nblk: (i + nblk, 0)),
            pl.BlockSpec((128, 128), lambda i: (0, 0)),
            pl.BlockSpec((1, 128), lambda i: (0, 0)),
            pl.BlockSpec((128, 64), lambda i: (0, 0)),
            pl.BlockSpec((1, 64), lambda i: (0, 0)),
            pl.BlockSpec((64, 64), lambda i: (0, 0)),
        ],
        out_specs=[
            pl.BlockSpec((BLK, 128), lambda i: (i, 0)),
            pl.BlockSpec((1, 64), lambda i: (0, 0)),
        ],
        out_shape=[
            jax.ShapeDtypeStruct((E, 128), jnp.float32),
            jax.ShapeDtypeStruct((1, 64), jnp.float32),
        ],
    )(G, G, W1, b1.reshape(1, 128), W2, b2.reshape(1, 64), H1e)


def _conv2_body(gs_ref, gd_ref, h1b_ref, h2w_ref, z_ref):
    gs = gs_ref[...]
    gd = gd_ref[...]
    h1b = h1b_ref[...]
    h2w = h2w_ref[...]
    zs = []
    for cs in (0, 64):
        h = jnp.tanh(0.5 * (gs[:, cs:cs + 64] + gd[:, cs:cs + 64]) + h1b)
        zs.append(jnp.dot(h, h2w, preferred_element_type=jnp.float32))
    z_ref[...] = jnp.concatenate(
        [zs[0], zs[1], jnp.zeros((gs.shape[0], 126), jnp.float32)], axis=1)


def _tc_conv2(G2, H1b, H2w, E):
    BLK = 2048
    nblk = E // BLK
    return pl.pallas_call(
        _conv2_body,
        grid=(nblk,),
        in_specs=[
            pl.BlockSpec((BLK, 128), lambda i: (i, 0)),
            pl.BlockSpec((BLK, 128), lambda i, nblk=nblk: (i + nblk, 0)),
            pl.BlockSpec((1, 64), lambda i: (0, 0)),
            pl.BlockSpec((64, 1), lambda i: (0, 0)),
        ],
        out_specs=pl.BlockSpec((BLK, 128), lambda i: (i, 0)),
        out_shape=jax.ShapeDtypeStruct((E, 128), jnp.float32),
    )(G2, G2, H1b.reshape(1, 64), H2w)


def _final_body(gs_ref, gd_ref, h2b_ref, w_ref):
    gs = gs_ref[...]
    gd = gd_ref[...]
    h2b = h2b_ref[0, 0]
    w0 = jax.nn.sigmoid(0.5 * (gs[:, 0] + gd[:, 0]) + h2b)
    w1 = jax.nn.sigmoid(0.5 * (gs[:, 1] + gd[:, 1]) + h2b)
    w_ref[...] = 0.5 * (w0 + w1)


def _tc_final(G3, H2b, E):
    BLK = 2048
    nblk = E // BLK
    return pl.pallas_call(
        _final_body,
        grid=(nblk,),
        in_specs=[
            pl.BlockSpec((BLK, 128), lambda i: (i, 0)),
            pl.BlockSpec((BLK, 128), lambda i, nblk=nblk: (i + nblk, 0)),
            pl.BlockSpec((1, 1), lambda i: (0, 0)),
        ],
        out_specs=pl.BlockSpec((BLK,), lambda i: (i,)),
        out_shape=jax.ShapeDtypeStruct((E,), jnp.float32),
    )(G3, G3, H2b.reshape(1, 1))


def kernel(x, edge_index, batch, t, W1, b1, W2, b2, H1w, H1b, H2w, H2b):
    N = x.shape[0]
    E = edge_index.shape[1]
    src, dst = edge_index[0], edge_index[1]

    adj = jnp.zeros((N, N), dtype=jnp.float32).at[src, dst].add(1.0)
    emb = [jax.lax.stop_gradient(_tsvd(adj, _SVD_DIM, _SVD_ITER, s))
           for s in _SVD_SEEDS]

    rowd = adj.sum(1)
    cold = adj.sum(0)
    Bdeg = rowd + cold
    Binv = jnp.where(Bdeg > 0, 1.0 / Bdeg, 0.0)

    Temb = jnp.concatenate(emb, axis=1)
    idx_all = jnp.concatenate([src, dst]).reshape(_NW, -1, 128)
    src3 = src.reshape(_NW, -1, 128)
    dst3 = dst.reshape(_NW, -1, 128)

    H1e, H1x = H1w[:64], H1w[64:]

    Gemb = _sc_gather(Temb, idx_all)
    U, ersum = _tc_mlp(Gemb, W1, b1, W2, b2, H1e, E)
    edge_pool = ersum / (2.0 * E)

    acc2sc = _sc_scatter(U, src3, dst3, jnp.zeros((N, 128), jnp.float32))
    xterm = 0.5 * ((rowd + cold)[:, None] * x + adj @ x + adj.T @ x) @ H1x
    e1 = Binv[:, None] * (acc2sc[0] + acc2sc[1]
                          + jnp.concatenate([xterm, xterm], axis=1))

    G2 = _sc_gather(e1, idx_all)
    Z = _tc_conv2(G2, H1b, H2w, E)
    accz = _sc_scatter(Z, src3, dst3, jnp.zeros((N, 128), jnp.float32))
    wtab = Binv[:, None] * (accz[0] + accz[1])

    G3 = _sc_gather(wtab, idx_all)
    weights = _tc_final(G3, H2b, E)
    return weights, edge_pool


# speedup vs baseline: 1.0815x; 1.0815x over previous
import functools

import jax
import jax.numpy as jnp
import numpy as np
from jax import lax
from jax.experimental import pallas as pl
from jax.experimental.pallas import tpu as pltpu
from jax.experimental.pallas import tpu_sc as plsc

_SVD_DIM = 64
_SVD_ITER = 5
_SVD_SEEDS = (0, 1)

_NC = 2
_NS = 16
_NW = _NC * _NS


def _tsvd(A, k, n_iter, seed):
    key = jax.random.key(seed)
    n = A.shape[1]
    Omega = jax.random.normal(key, (n, k + 10), dtype=A.dtype)
    Y = A @ Omega
    for _ in range(n_iter):
        Y = A @ (A.T @ Y)
    Q, _ = jnp.linalg.qr(Y)
    Bm = Q.T @ A
    Ub, s, Vt = jnp.linalg.svd(Bm, full_matrices=False)
    U = Q @ Ub
    return U[:, :k] * s[:k]


def _sc_gather(table, idx3):
    NW, NB, BT = idx3.shape
    N, D = table.shape
    M = NW * NB * BT
    mesh = plsc.VectorSubcoreMesh(core_axis_name="c", subcore_axis_name="s")

    @functools.partial(
        pl.kernel,
        out_type=jax.ShapeDtypeStruct((M, D), jnp.float32),
        mesh=mesh,
        scratch_types=[
            pltpu.VMEM((NB, BT), jnp.int32),
            pltpu.VMEM((BT, D), jnp.float32),
            pltpu.SemaphoreType.DMA,
        ],
    )
    def k(table_hbm, idx_hbm, out_hbm, idx_v, rows_v, sem):
        c = lax.axis_index("c")
        s = lax.axis_index("s")
        wid = s * _NC + c
        base = wid * (NB * BT)
        pltpu.sync_copy(idx_hbm.at[wid], idx_v)

        def body(j, carry):
            pltpu.async_copy(table_hbm.at[idx_v.at[j]], rows_v, sem).wait()
            pltpu.sync_copy(rows_v, out_hbm.at[pl.ds(base + j * BT, BT)])
            return carry

        lax.fori_loop(0, NB, body, 0)

    return k(table, idx3)


def _sc_scatter(vals, src3, dst3, zeros):
    NW, NB, BT = src3.shape
    E, D = vals.shape
    N = zeros.shape[0]
    RPW = N // _NS
    mesh = plsc.VectorSubcoreMesh(core_axis_name="c", subcore_axis_name="s")

    @functools.partial(
        pl.kernel,
        out_type=jax.ShapeDtypeStruct((_NC, N, D), jnp.float32),
        mesh=mesh,
        scratch_types=[
            pltpu.VMEM((NB, BT), jnp.int32),
            pltpu.VMEM((NB, BT), jnp.int32),
            pltpu.VMEM((BT, D), jnp.float32),
            pltpu.VMEM_SHARED((N, D), jnp.float32),
        ],
    )
    def k(vals_hbm, src_hbm, dst_hbm, zeros_hbm, out_hbm, idxs_v, idxd_v,
          rows_v, acc):
        c = lax.axis_index("c")
        s = lax.axis_index("s")
        wid = s * _NC + c
        pltpu.sync_copy(zeros_hbm.at[pl.ds(s * RPW, RPW)],
                        acc.at[pl.ds(s * RPW, RPW)])
        pltpu.sync_copy(src_hbm.at[wid], idxs_v)
        pltpu.sync_copy(dst_hbm.at[wid], idxd_v)
        plsc.subcore_barrier()
        base = wid * (NB * BT)

        def body(j, carry):
            pltpu.sync_copy(vals_hbm.at[pl.ds(base + j * BT, BT)], rows_v)
            pltpu.sync_copy(rows_v, acc.at[idxs_v.at[j]], add=True)
            pltpu.sync_copy(rows_v, acc.at[idxd_v.at[j]], add=True)
            return carry

        lax.fori_loop(0, NB, body, 0)
        plsc.subcore_barrier()
        pltpu.sync_copy(acc.at[pl.ds(s * RPW, RPW)],
                        out_hbm.at[c, pl.ds(s * RPW, RPW)])

    return k(vals, src3, dst3, zeros)


def _mlp_body(gs_ref, gd_ref, w1_ref, b1_ref, w2_ref, b2_ref, h1e_ref,
              u_ref, ersum_ref):
    gs = gs_ref[...]
    gd = gd_ref[...]
    w1a = w1_ref[0:64, :]
    w1b = w1_ref[64:128, :]
    b1 = b1_ref[...]
    w2 = w2_ref[...]
    b2 = b2_ref[...]
    h1e = h1e_ref[...]

    def seedpart(cs):
        h = jnp.maximum(
            jnp.dot(gs[:, cs:cs + 64], w1a, preferred_element_type=jnp.float32)
            + jnp.dot(gd[:, cs:cs + 64], w1b, preferred_element_type=jnp.float32)
            + b1, 0.0)
        return jnp.dot(h, w2, preferred_element_type=jnp.float32) + b2

    er0 = seedpart(0)
    er1 = seedpart(64)
    u0 = jnp.dot(er0, h1e, preferred_element_type=jnp.float32)
    u1 = jnp.dot(er1, h1e, preferred_element_type=jnp.float32)
    u_ref[...] = jnp.concatenate([u0, u1], axis=1)

    @pl.when(pl.program_id(0) == 0)
    def _():
        ersum_ref[...] = jnp.zeros_like(ersum_ref)

    ersum_ref[...] += jnp.sum(er0 + er1, axis=0, keepdims=True)


def _tc_mlp(G, W1, b1, W2, b2, H1e, E):
    BLK = 2048
    nblk = E // BLK
    return pl.pallas_call(
        _mlp_body,
        grid=(nblk,),
        in_specs=[
            pl.BlockSpec((BLK, 128), lambda i: (i, 0)),
            pl.BlockSpec((BLK, 128), lambda i, nblk=nblk: (i + nblk, 0)),
            pl.BlockSpec((128, 128), lambda i: (0, 0)),
            pl.BlockSpec((1, 128), lambda i: (0, 0)),
            pl.BlockSpec((128, 64), lambda i: (0, 0)),
            pl.BlockSpec((1, 64), lambda i: (0, 0)),
            pl.BlockSpec((64, 64), lambda i: (0, 0)),
        ],
        out_specs=[
            pl.BlockSpec((BLK, 128), lambda i: (i, 0)),
            pl.BlockSpec((1, 64), lambda i: (0, 0)),
        ],
        out_shape=[
            jax.ShapeDtypeStruct((E, 128), jnp.float32),
            jax.ShapeDtypeStruct((1, 64), jnp.float32),
        ],
    )(G, G, W1, b1.reshape(1, 128), W2, b2.reshape(1, 64), H1e)


def _conv2_body(gs_ref, gd_ref, h1b_ref, h2w_ref, z0_ref, z1_ref):
    gs = gs_ref[...]
    gd = gd_ref[...]
    h1b = h1b_ref[...]
    h2w = h2w_ref[...]
    zs = []
    for cs in (0, 64):
        h = jnp.tanh(0.5 * (gs[:, cs:cs + 64] + gd[:, cs:cs + 64]) + h1b)
        zs.append(jnp.dot(h, h2w, preferred_element_type=jnp.float32))
    z0_ref[...] = zs[0][:, 0]
    z1_ref[...] = zs[1][:, 0]


def _tc_conv2(G2, H1b, H2w, E):
    BLK = 2048
    nblk = E // BLK
    return pl.pallas_call(
        _conv2_body,
        grid=(nblk,),
        in_specs=[
            pl.BlockSpec((BLK, 128), lambda i: (i, 0)),
            pl.BlockSpec((BLK, 128), lambda i, nblk=nblk: (i + nblk, 0)),
            pl.BlockSpec((1, 64), lambda i: (0, 0)),
            pl.BlockSpec((64, 1), lambda i: (0, 0)),
        ],
        out_specs=[
            pl.BlockSpec((BLK,), lambda i: (i,)),
            pl.BlockSpec((BLK,), lambda i: (i,)),
        ],
        out_shape=[
            jax.ShapeDtypeStruct((E,), jnp.float32),
            jax.ShapeDtypeStruct((E,), jnp.float32),
        ],
    )(G2, G2, H1b.reshape(1, 64), H2w)


def _sc_scatter_scalar2(z0, z1, src_f, dst_f, N, zeros2):
    E = z0.shape[0]
    EPW = E // _NW
    NG = EPW // 16
    NL = N * 16
    mesh = plsc.VectorSubcoreMesh(core_axis_name="c", subcore_axis_name="s")

    @functools.partial(
        pl.kernel,
        out_type=jax.ShapeDtypeStruct((_NW, 2, NL), jnp.float32),
        mesh=mesh,
        compiler_params=pltpu.CompilerParams(needs_layout_passes=False),
        scratch_types=[
            pltpu.VMEM((EPW,), jnp.int32),
            pltpu.VMEM((EPW,), jnp.int32),
            pltpu.VMEM((EPW,), jnp.float32),
            pltpu.VMEM((EPW,), jnp.float32),
            pltpu.VMEM((NL,), jnp.float32),
            pltpu.VMEM((NL,), jnp.float32),
        ],
    )
    def k(z0_hbm, z1_hbm, src_hbm, dst_hbm, zeros_hbm, out_hbm,
          s_v, d_v, z0_v, z1_v, acc0, acc1):
        c = lax.axis_index("c")
        s = lax.axis_index("s")
        wid = s * _NC + c
        base = wid * EPW
        pltpu.sync_copy(src_hbm.at[pl.ds(base, EPW)], s_v)
        pltpu.sync_copy(dst_hbm.at[pl.ds(base, EPW)], d_v)
        pltpu.sync_copy(z0_hbm.at[pl.ds(base, EPW)], z0_v)
        pltpu.sync_copy(z1_hbm.at[pl.ds(base, EPW)], z1_v)
        pltpu.sync_copy(zeros_hbm, acc0)
        pltpu.sync_copy(zeros_hbm, acc1)
        lanes = lax.iota(jnp.int32, 16)

        def body(g, carry):
            sv = s_v[pl.ds(g * 16, 16)] * 16 + lanes
            dv = d_v[pl.ds(g * 16, 16)] * 16 + lanes
            z0g = z0_v[pl.ds(g * 16, 16)]
            z1g = z1_v[pl.ds(g * 16, 16)]
            plsc.addupdate_scatter(acc0, [sv], z0g)
            plsc.addupdate_scatter(acc0, [dv], z0g)
            plsc.addupdate_scatter(acc1, [sv], z1g)
            plsc.addupdate_scatter(acc1, [dv], z1g)
            return carry

        lax.fori_loop(0, NG, body, 0)
        pltpu.sync_copy(acc0, out_hbm.at[wid, 0])
        pltpu.sync_copy(acc1, out_hbm.at[wid, 1])

    return k(z0, z1, src_f, dst_f, zeros2)


def _sc_finale(wtab_flat, src_f, dst_f, h2b16):
    E = src_f.shape[0]
    EPW = E // _NW
    NG = EPW // 16
    TN = wtab_flat.shape[0]
    mesh = plsc.VectorSubcoreMesh(core_axis_name="c", subcore_axis_name="s")

    @functools.partial(
        pl.kernel,
        out_type=jax.ShapeDtypeStruct((E,), jnp.float32),
        mesh=mesh,
        compiler_params=pltpu.CompilerParams(needs_layout_passes=False),
        scratch_types=[
            pltpu.VMEM((TN,), jnp.float32),
            pltpu.VMEM((EPW,), jnp.int32),
            pltpu.VMEM((EPW,), jnp.int32),
            pltpu.VMEM((EPW,), jnp.float32),
            pltpu.VMEM((16,), jnp.float32),
        ],
    )
    def k(wt_hbm, src_hbm, dst_hbm, h2b_hbm, out_hbm, wt_v, s_v, d_v, o_v,
          h2b_v):
        c = lax.axis_index("c")
        s = lax.axis_index("s")
        wid = s * _NC + c
        base = wid * EPW
        pltpu.sync_copy(wt_hbm, wt_v)
        pltpu.sync_copy(h2b_hbm, h2b_v)
        pltpu.sync_copy(src_hbm.at[pl.ds(base, EPW)], s_v)
        pltpu.sync_copy(dst_hbm.at[pl.ds(base, EPW)], d_v)

        def body(g, carry):
            sv = s_v[pl.ds(g * 16, 16)] * 2
            dv = d_v[pl.ds(g * 16, 16)] * 2
            h2b = h2b_v[...]
            a0 = plsc.load_gather(wt_v, [sv])
            b0 = plsc.load_gather(wt_v, [dv])
            a1 = plsc.load_gather(wt_v, [sv + 1])
            b1 = plsc.load_gather(wt_v, [dv + 1])
            q0 = 0.5 * (a0 + b0) + h2b
            q1 = 0.5 * (a1 + b1) + h2b
            w0 = 1.0 / (1.0 + jnp.exp(-q0))
            w1 = 1.0 / (1.0 + jnp.exp(-q1))
            o_v[pl.ds(g * 16, 16)] = 0.5 * (w0 + w1)
            return carry

        lax.fori_loop(0, NG, body, 0)
        pltpu.sync_copy(o_v, out_hbm.at[pl.ds(base, EPW)])

    return k(wtab_flat, src_f, dst_f, h2b16)


def kernel(x, edge_index, batch, t, W1, b1, W2, b2, H1w, H1b, H2w, H2b):
    N = x.shape[0]
    E = edge_index.shape[1]
    src, dst = edge_index[0], edge_index[1]

    adj = jnp.zeros((N, N), dtype=jnp.float32).at[src, dst].add(1.0)
    emb = [jax.lax.stop_gradient(_tsvd(adj, _SVD_DIM, _SVD_ITER, s))
           for s in _SVD_SEEDS]

    rowd = adj.sum(1)
    cold = adj.sum(0)
    Bdeg = rowd + cold
    Binv = jnp.where(Bdeg > 0, 1.0 / Bdeg, 0.0)

    Temb = jnp.concatenate(emb, axis=1)
    idx_all = jnp.concatenate([src, dst]).reshape(_NW, -1, 128)
    src3 = src.reshape(_NW, -1, 128)
    dst3 = dst.reshape(_NW, -1, 128)

    H1e, H1x = H1w[:64], H1w[64:]

    Gemb = _sc_gather(Temb, idx_all)
    U, ersum = _tc_mlp(Gemb, W1, b1, W2, b2, H1e, E)
    edge_pool = ersum / (2.0 * E)

    acc2sc = _sc_scatter(U, src3, dst3, jnp.zeros((N, 128), jnp.float32))
    xterm = 0.5 * ((rowd + cold)[:, None] * x + adj @ x + adj.T @ x) @ H1x
    e1 = Binv[:, None] * (acc2sc[0] + acc2sc[1]
                          + jnp.concatenate([xterm, xterm], axis=1))

    G2 = _sc_gather(e1, idx_all)
    z0, z1 = _tc_conv2(G2, H1b, H2w, E)
    zacc = _sc_scatter_scalar2(z0, z1, src, dst, N,
                               jnp.zeros((N * 16,), jnp.float32))
    zsum = zacc.sum(axis=0).reshape(2, N, 16).sum(axis=2)
    wtab_flat = (zsum * Binv[None, :]).T.reshape(-1)

    weights = _sc_finale(wtab_flat, src, dst,
                         jnp.full((16,), H2b[0], jnp.float32))
    return weights, edge_pool


# speedup vs baseline: 1.1298x; 1.0447x over previous
import functools

import jax
import jax.numpy as jnp
import numpy as np
from jax import lax
from jax.experimental import pallas as pl
from jax.experimental.pallas import tpu as pltpu
from jax.experimental.pallas import tpu_sc as plsc

_SVD_DIM = 64
_SVD_ITER = 5
_SVD_SEEDS = (0, 1)

_NC = 2
_NS = 16
_NW = _NC * _NS


def _tsvd(A, k, n_iter, seed):
    key = jax.random.key(seed)
    n = A.shape[1]
    Omega = jax.random.normal(key, (n, k + 10), dtype=A.dtype)
    Y = A @ Omega
    for _ in range(n_iter):
        Y = A @ (A.T @ Y)
    Q, _ = jnp.linalg.qr(Y)
    Bm = Q.T @ A
    Ub, s, Vt = jnp.linalg.svd(Bm, full_matrices=False)
    U = Q @ Ub
    return U[:, :k] * s[:k]


_DIMN_T = (((0,), (0,)), ((), ()))


def _power_body(a_ref, om0_ref, om1_ref, x_ref, h1x_ref,
                y0_ref, y1_ref, aux_ref):
    A = a_ref[...]
    for om_ref, y_ref in ((om0_ref, y0_ref), (om1_ref, y1_ref)):
        Y = jnp.dot(A, om_ref[...], preferred_element_type=jnp.float32)
        for _ in range(_SVD_ITER):
            Z = lax.dot_general(A, Y, _DIMN_T,
                                preferred_element_type=jnp.float32)
            Y = jnp.dot(A, Z, preferred_element_type=jnp.float32)
        y_ref[...] = Y
    xv = x_ref[...]
    ax = jnp.dot(A, xv, preferred_element_type=jnp.float32)
    atx = lax.dot_general(A, xv, _DIMN_T, preferred_element_type=jnp.float32)
    rs = jnp.sum(A, axis=1)
    cs = jnp.sum(A, axis=0)
    xt = jnp.dot(0.5 * ((rs + cs)[:, None] * xv + ax + atx), h1x_ref[...],
                 preferred_element_type=jnp.float32)
    aux_ref[...] = jnp.concatenate(
        [xt, rs[:, None], cs[:, None],
         jnp.zeros((A.shape[0], 62), jnp.float32)], axis=1)


def _tc_power_aux(adj, Om0p, Om1p, x, H1x):
    N = adj.shape[0]
    return pl.pallas_call(
        _power_body,
        out_shape=[
            jax.ShapeDtypeStruct((N, 128), jnp.float32),
            jax.ShapeDtypeStruct((N, 128), jnp.float32),
            jax.ShapeDtypeStruct((N, 128), jnp.float32),
        ],
    )(adj, Om0p, Om1p, x, H1x)


def _qta_body(q0_ref, q1_ref, a_ref, b0_ref, b1_ref):
    A = a_ref[...]
    b0_ref[...] = lax.dot_general(q0_ref[...], A, _DIMN_T,
                                  preferred_element_type=jnp.float32)
    b1_ref[...] = lax.dot_general(q1_ref[...], A, _DIMN_T,
                                  preferred_element_type=jnp.float32)


def _tc_qta(Q0p, Q1p, adj):
    N = adj.shape[0]
    K = Q0p.shape[1]
    return pl.pallas_call(
        _qta_body,
        out_shape=[
            jax.ShapeDtypeStruct((K, N), jnp.float32),
            jax.ShapeDtypeStruct((K, N), jnp.float32),
        ],
    )(Q0p, Q1p, adj)


def _sc_gather(table, idx3):
    NW, NB, BT = idx3.shape
    N, D = table.shape
    M = NW * NB * BT
    mesh = plsc.VectorSubcoreMesh(core_axis_name="c", subcore_axis_name="s")

    @functools.partial(
        pl.kernel,
        out_type=jax.ShapeDtypeStruct((M, D), jnp.float32),
        mesh=mesh,
        scratch_types=[
            pltpu.VMEM((NB, BT), jnp.int32),
            pltpu.VMEM((BT, D), jnp.float32),
            pltpu.SemaphoreType.DMA,
        ],
    )
    def k(table_hbm, idx_hbm, out_hbm, idx_v, rows_v, sem):
        c = lax.axis_index("c")
        s = lax.axis_index("s")
        wid = s * _NC + c
        base = wid * (NB * BT)
        pltpu.sync_copy(idx_hbm.at[wid], idx_v)

        def body(j, carry):
            pltpu.async_copy(table_hbm.at[idx_v.at[j]], rows_v, sem).wait()
            pltpu.sync_copy(rows_v, out_hbm.at[pl.ds(base + j * BT, BT)])
            return carry

        lax.fori_loop(0, NB, body, 0)

    return k(table, idx3)


def _sc_scatter(vals, src3, dst3, zeros):
    NW, NB, BT = src3.shape
    E, D = vals.shape
    N = zeros.shape[0]
    RPW = N // _NS
    mesh = plsc.VectorSubcoreMesh(core_axis_name="c", subcore_axis_name="s")

    @functools.partial(
        pl.kernel,
        out_type=jax.ShapeDtypeStruct((_NC, N, D), jnp.float32),
        mesh=mesh,
        scratch_types=[
            pltpu.VMEM((NB, BT), jnp.int32),
            pltpu.VMEM((NB, BT), jnp.int32),
            pltpu.VMEM((BT, D), jnp.float32),
            pltpu.VMEM_SHARED((N, D), jnp.float32),
        ],
    )
    def k(vals_hbm, src_hbm, dst_hbm, zeros_hbm, out_hbm, idxs_v, idxd_v,
          rows_v, acc):
        c = lax.axis_index("c")
        s = lax.axis_index("s")
        wid = s * _NC + c
        pltpu.sync_copy(zeros_hbm.at[pl.ds(s * RPW, RPW)],
                        acc.at[pl.ds(s * RPW, RPW)])
        pltpu.sync_copy(src_hbm.at[wid], idxs_v)
        pltpu.sync_copy(dst_hbm.at[wid], idxd_v)
        plsc.subcore_barrier()
        base = wid * (NB * BT)

        def body(j, carry):
            pltpu.sync_copy(vals_hbm.at[pl.ds(base + j * BT, BT)], rows_v)
            pltpu.sync_copy(rows_v, acc.at[idxs_v.at[j]], add=True)
            pltpu.sync_copy(rows_v, acc.at[idxd_v.at[j]], add=True)
            return carry

        lax.fori_loop(0, NB, body, 0)
        plsc.subcore_barrier()
        pltpu.sync_copy(acc.at[pl.ds(s * RPW, RPW)],
                        out_hbm.at[c, pl.ds(s * RPW, RPW)])

    return k(vals, src3, dst3, zeros)


def _mlp_body(gs_ref, gd_ref, w1_ref, b1_ref, w2_ref, b2_ref, h1e_ref,
              u_ref, ersum_ref):
    gs = gs_ref[...]
    gd = gd_ref[...]
    w1a = w1_ref[0:64, :]
    w1b = w1_ref[64:128, :]
    b1 = b1_ref[...]
    w2 = w2_ref[...]
    b2 = b2_ref[...]
    h1e = h1e_ref[...]

    def seedpart(cs):
        h = jnp.maximum(
            jnp.dot(gs[:, cs:cs + 64], w1a, preferred_element_type=jnp.float32)
            + jnp.dot(gd[:, cs:cs + 64], w1b, preferred_element_type=jnp.float32)
            + b1, 0.0)
        return jnp.dot(h, w2, preferred_element_type=jnp.float32) + b2

    er0 = seedpart(0)
    er1 = seedpart(64)
    u0 = jnp.dot(er0, h1e, preferred_element_type=jnp.float32)
    u1 = jnp.dot(er1, h1e, preferred_element_type=jnp.float32)
    u_ref[...] = jnp.concatenate([u0, u1], axis=1)

    @pl.when(pl.program_id(0) == 0)
    def _():
        ersum_ref[...] = jnp.zeros_like(ersum_ref)

    ersum_ref[...] += jnp.sum(er0 + er1, axis=0, keepdims=True)


def _tc_mlp(G, W1, b1, W2, b2, H1e, E):
    BLK = 2048
    nblk = E // BLK
    return pl.pallas_call(
        _mlp_body,
        grid=(nblk,),
        in_specs=[
            pl.BlockSpec((BLK, 128), lambda i: (i, 0)),
            pl.BlockSpec((BLK, 128), lambda i, nblk=nblk: (i + nblk, 0)),
            pl.BlockSpec((128, 128), lambda i: (0, 0)),
            pl.BlockSpec((1, 128), lambda i: (0, 0)),
            pl.BlockSpec((128, 64), lambda i: (0, 0)),
            pl.BlockSpec((1, 64), lambda i: (0, 0)),
            pl.BlockSpec((64, 64), lambda i: (0, 0)),
        ],
        out_specs=[
            pl.BlockSpec((BLK, 128), lambda i: (i, 0)),
            pl.BlockSpec((1, 64), lambda i: (0, 0)),
        ],
        out_shape=[
            jax.ShapeDtypeStruct((E, 128), jnp.float32),
            jax.ShapeDtypeStruct((1, 64), jnp.float32),
        ],
    )(G, G, W1, b1.reshape(1, 128), W2, b2.reshape(1, 64), H1e)


def _conv2_body(gs_ref, gd_ref, h1b_ref, h2w_ref, z0_ref, z1_ref):
    gs = gs_ref[...]
    gd = gd_ref[...]
    h1b = h1b_ref[...]
    h2w = h2w_ref[...]
    zs = []
    for cs in (0, 64):
        h = jnp.tanh(0.5 * (gs[:, cs:cs + 64] + gd[:, cs:cs + 64]) + h1b)
        zs.append(jnp.dot(h, h2w, preferred_element_type=jnp.float32))
    z0_ref[...] = zs[0][:, 0]
    z1_ref[...] = zs[1][:, 0]


def _tc_conv2(G2, H1b, H2w, E):
    BLK = 2048
    nblk = E // BLK
    return pl.pallas_call(
        _conv2_body,
        grid=(nblk,),
        in_specs=[
            pl.BlockSpec((BLK, 128), lambda i: (i, 0)),
            pl.BlockSpec((BLK, 128), lambda i, nblk=nblk: (i + nblk, 0)),
            pl.BlockSpec((1, 64), lambda i: (0, 0)),
            pl.BlockSpec((64, 1), lambda i: (0, 0)),
        ],
        out_specs=[
            pl.BlockSpec((BLK,), lambda i: (i,)),
            pl.BlockSpec((BLK,), lambda i: (i,)),
        ],
        out_shape=[
            jax.ShapeDtypeStruct((E,), jnp.float32),
            jax.ShapeDtypeStruct((E,), jnp.float32),
        ],
    )(G2, G2, H1b.reshape(1, 64), H2w)


def _sc_scatter_scalar2(z0, z1, src_f, dst_f, N, zeros2):
    E = z0.shape[0]
    EPW = E // _NW
    NG = EPW // 16
    NL = N * 16
    mesh = plsc.VectorSubcoreMesh(core_axis_name="c", subcore_axis_name="s")

    @functools.partial(
        pl.kernel,
        out_type=jax.ShapeDtypeStruct((_NW, 2, NL), jnp.float32),
        mesh=mesh,
        compiler_params=pltpu.CompilerParams(needs_layout_passes=False),
        scratch_types=[
            pltpu.VMEM((EPW,), jnp.int32),
            pltpu.VMEM((EPW,), jnp.int32),
            pltpu.VMEM((EPW,), jnp.float32),
            pltpu.VMEM((EPW,), jnp.float32),
            pltpu.VMEM((NL,), jnp.float32),
            pltpu.VMEM((NL,), jnp.float32),
        ],
    )
    def k(z0_hbm, z1_hbm, src_hbm, dst_hbm, zeros_hbm, out_hbm,
          s_v, d_v, z0_v, z1_v, acc0, acc1):
        c = lax.axis_index("c")
        s = lax.axis_index("s")
        wid = s * _NC + c
        base = wid * EPW
        pltpu.sync_copy(src_hbm.at[pl.ds(base, EPW)], s_v)
        pltpu.sync_copy(dst_hbm.at[pl.ds(base, EPW)], d_v)
        pltpu.sync_copy(z0_hbm.at[pl.ds(base, EPW)], z0_v)
        pltpu.sync_copy(z1_hbm.at[pl.ds(base, EPW)], z1_v)
        pltpu.sync_copy(zeros_hbm, acc0)
        pltpu.sync_copy(zeros_hbm, acc1)
        lanes = lax.iota(jnp.int32, 16)

        def body(g, carry):
            sv = s_v[pl.ds(g * 16, 16)] * 16 + lanes
            dv = d_v[pl.ds(g * 16, 16)] * 16 + lanes
            z0g = z0_v[pl.ds(g * 16, 16)]
            z1g = z1_v[pl.ds(g * 16, 16)]
            plsc.addupdate_scatter(acc0, [sv], z0g)
            plsc.addupdate_scatter(acc0, [dv], z0g)
            plsc.addupdate_scatter(acc1, [sv], z1g)
            plsc.addupdate_scatter(acc1, [dv], z1g)
            return carry

        lax.fori_loop(0, NG, body, 0)
        pltpu.sync_copy(acc0, out_hbm.at[wid, 0])
        pltpu.sync_copy(acc1, out_hbm.at[wid, 1])

    return k(z0, z1, src_f, dst_f, zeros2)


def _sc_finale(wtab_flat, src_f, dst_f, h2b16):
    E = src_f.shape[0]
    EPW = E // _NW
    NG = EPW // 16
    TN = wtab_flat.shape[0]
    mesh = plsc.VectorSubcoreMesh(core_axis_name="c", subcore_axis_name="s")

    @functools.partial(
        pl.kernel,
        out_type=jax.ShapeDtypeStruct((E,), jnp.float32),
        mesh=mesh,
        compiler_params=pltpu.CompilerParams(needs_layout_passes=False),
        scratch_types=[
            pltpu.VMEM((TN,), jnp.float32),
            pltpu.VMEM((EPW,), jnp.int32),
            pltpu.VMEM((EPW,), jnp.int32),
            pltpu.VMEM((EPW,), jnp.float32),
            pltpu.VMEM((16,), jnp.float32),
        ],
    )
    def k(wt_hbm, src_hbm, dst_hbm, h2b_hbm, out_hbm, wt_v, s_v, d_v, o_v,
          h2b_v):
        c = lax.axis_index("c")
        s = lax.axis_index("s")
        wid = s * _NC + c
        base = wid * EPW
        pltpu.sync_copy(wt_hbm, wt_v)
        pltpu.sync_copy(h2b_hbm, h2b_v)
        pltpu.sync_copy(src_hbm.at[pl.ds(base, EPW)], s_v)
        pltpu.sync_copy(dst_hbm.at[pl.ds(base, EPW)], d_v)

        def body(g, carry):
            sv = s_v[pl.ds(g * 16, 16)] * 2
            dv = d_v[pl.ds(g * 16, 16)] * 2
            h2b = h2b_v[...]
            a0 = plsc.load_gather(wt_v, [sv])
            b0 = plsc.load_gather(wt_v, [dv])
            a1 = plsc.load_gather(wt_v, [sv + 1])
            b1 = plsc.load_gather(wt_v, [dv + 1])
            q0 = 0.5 * (a0 + b0) + h2b
            q1 = 0.5 * (a1 + b1) + h2b
            w0 = 1.0 / (1.0 + jnp.exp(-q0))
            w1 = 1.0 / (1.0 + jnp.exp(-q1))
            o_v[pl.ds(g * 16, 16)] = 0.5 * (w0 + w1)
            return carry

        lax.fori_loop(0, NG, body, 0)
        pltpu.sync_copy(o_v, out_hbm.at[pl.ds(base, EPW)])

    return k(wtab_flat, src_f, dst_f, h2b16)


def kernel(x, edge_index, batch, t, W1, b1, W2, b2, H1w, H1b, H2w, H2b):
    N = x.shape[0]
    E = edge_index.shape[1]
    src, dst = edge_index[0], edge_index[1]

    adj = jnp.zeros((N, N), dtype=jnp.float32).at[src, dst].add(1.0)

    H1e, H1x = H1w[:64], H1w[64:]
    K = _SVD_DIM + 10
    Oms = []
    for s in _SVD_SEEDS:
        Om = jax.random.normal(jax.random.key(s), (N, K), dtype=jnp.float32)
        Oms.append(jnp.pad(Om, ((0, 0), (0, 128 - K))))
    Y0p, Y1p, aux = _tc_power_aux(adj, Oms[0], Oms[1], x, H1x)
    xterm = aux[:, :64]
    rowd, cold = aux[:, 64], aux[:, 65]
    Bdeg = rowd + cold
    Binv = jnp.where(Bdeg > 0, 1.0 / Bdeg, 0.0)

    Qs = [jnp.linalg.qr(Yp[:, :K])[0] for Yp in (Y0p, Y1p)]
    B0, B1 = _tc_qta(jnp.pad(Qs[0], ((0, 0), (0, 128 - K))),
                     jnp.pad(Qs[1], ((0, 0), (0, 128 - K))), adj)
    emb = []
    for Q, Bp in zip(Qs, (B0, B1)):
        Ub, sv, _ = jnp.linalg.svd(Bp[:K], full_matrices=False)
        emb.append(jax.lax.stop_gradient(
            Q @ (Ub[:, :_SVD_DIM] * sv[:_SVD_DIM])))

    Temb = jnp.concatenate(emb, axis=1)
    idx_all = jnp.concatenate([src, dst]).reshape(_NW, -1, 128)
    src3 = src.reshape(_NW, -1, 128)
    dst3 = dst.reshape(_NW, -1, 128)

    Gemb = _sc_gather(Temb, idx_all)
    U, ersum = _tc_mlp(Gemb, W1, b1, W2, b2, H1e, E)
    edge_pool = ersum / (2.0 * E)

    acc2sc = _sc_scatter(U, src3, dst3, jnp.zeros((N, 128), jnp.float32))
    e1 = Binv[:, None] * (acc2sc[0] + acc2sc[1]
                          + jnp.concatenate([xterm, xterm], axis=1))

    G2 = _sc_gather(e1, idx_all)
    z0, z1 = _tc_conv2(G2, H1b, H2w, E)
    zacc = _sc_scatter_scalar2(z0, z1, src, dst, N,
                               jnp.zeros((N * 16,), jnp.float32))
    zsum = zacc.sum(axis=0).reshape(2, N, 16).sum(axis=2)
    wtab_flat = (zsum * Binv[None, :]).T.reshape(-1)

    weights = _sc_finale(wtab_flat, src, dst,
                         jnp.full((16,), H2b[0], jnp.float32))
    return weights, edge_pool


# speedup vs baseline: 1.1629x; 1.0293x over previous
import functools

import jax
import jax.numpy as jnp
import numpy as np
from jax import lax
from jax.experimental import pallas as pl
from jax.experimental.pallas import tpu as pltpu
from jax.experimental.pallas import tpu_sc as plsc

_SVD_DIM = 64
_SVD_ITER = 5
_SVD_SEEDS = (0, 1)

_NC = 2
_NS = 16
_NW = _NC * _NS


def _tsvd(A, k, n_iter, seed):
    key = jax.random.key(seed)
    n = A.shape[1]
    Omega = jax.random.normal(key, (n, k + 10), dtype=A.dtype)
    Y = A @ Omega
    for _ in range(n_iter):
        Y = A @ (A.T @ Y)
    Q, _ = jnp.linalg.qr(Y)
    Bm = Q.T @ A
    Ub, s, Vt = jnp.linalg.svd(Bm, full_matrices=False)
    U = Q @ Ub
    return U[:, :k] * s[:k]


_DIMN_T = (((0,), (0,)), ((), ()))


def _power_body(a_ref, om0_ref, om1_ref, x_ref, h1x_ref,
                y0_ref, y1_ref, aux_ref):
    A = a_ref[...]
    for om_ref, y_ref in ((om0_ref, y0_ref), (om1_ref, y1_ref)):
        Y = jnp.dot(A, om_ref[...], preferred_element_type=jnp.float32)
        for _ in range(_SVD_ITER):
            Z = lax.dot_general(A, Y, _DIMN_T,
                                preferred_element_type=jnp.float32)
            Y = jnp.dot(A, Z, preferred_element_type=jnp.float32)
        y_ref[...] = Y
    xv = x_ref[...]
    ax = jnp.dot(A, xv, preferred_element_type=jnp.float32)
    atx = lax.dot_general(A, xv, _DIMN_T, preferred_element_type=jnp.float32)
    rs = jnp.sum(A, axis=1)
    cs = jnp.sum(A, axis=0)
    xt = jnp.dot(0.5 * ((rs + cs)[:, None] * xv + ax + atx), h1x_ref[...],
                 preferred_element_type=jnp.float32)
    aux_ref[...] = jnp.concatenate(
        [xt, rs[:, None], cs[:, None],
         jnp.zeros((A.shape[0], 62), jnp.float32)], axis=1)


def _tc_power_aux(adj, Om0p, Om1p, x, H1x):
    N = adj.shape[0]
    return pl.pallas_call(
        _power_body,
        out_shape=[
            jax.ShapeDtypeStruct((N, 128), jnp.float32),
            jax.ShapeDtypeStruct((N, 128), jnp.float32),
            jax.ShapeDtypeStruct((N, 128), jnp.float32),
        ],
    )(adj, Om0p, Om1p, x, H1x)


def _qta_body(q0_ref, q1_ref, a_ref, b0_ref, b1_ref):
    A = a_ref[...]
    b0_ref[...] = lax.dot_general(q0_ref[...], A, _DIMN_T,
                                  preferred_element_type=jnp.float32)
    b1_ref[...] = lax.dot_general(q1_ref[...], A, _DIMN_T,
                                  preferred_element_type=jnp.float32)


def _tc_qta(Q0p, Q1p, adj):
    N = adj.shape[0]
    K = Q0p.shape[1]
    return pl.pallas_call(
        _qta_body,
        out_shape=[
            jax.ShapeDtypeStruct((K, N), jnp.float32),
            jax.ShapeDtypeStruct((K, N), jnp.float32),
        ],
    )(Q0p, Q1p, adj)


def _sc_gather(table, idx3):
    NW, NB, BT = idx3.shape
    N, D = table.shape
    M = NW * NB * BT
    mesh = plsc.VectorSubcoreMesh(core_axis_name="c", subcore_axis_name="s")

    @functools.partial(
        pl.kernel,
        out_type=jax.ShapeDtypeStruct((M, D), jnp.float32),
        mesh=mesh,
        scratch_types=[
            pltpu.VMEM((NB, BT), jnp.int32),
            pltpu.VMEM((BT, D), jnp.float32),
            pltpu.VMEM((BT, D), jnp.float32),
            pltpu.SemaphoreType.DMA,
            pltpu.SemaphoreType.DMA,
        ],
    )
    def k(table_hbm, idx_hbm, out_hbm, idx_v, rows0, rows1, sem0, sem1):
        c = lax.axis_index("c")
        s = lax.axis_index("s")
        wid = s * _NC + c
        base = wid * (NB * BT)
        pltpu.sync_copy(idx_hbm.at[wid], idx_v)
        pltpu.async_copy(table_hbm.at[idx_v.at[0]], rows0, sem0)

        def body(j2, carry):
            j = j2 * 2
            pltpu.async_copy(table_hbm.at[idx_v.at[j + 1]], rows1, sem1)
            pltpu.make_async_copy(table_hbm.at[idx_v.at[j]], rows0, sem0).wait()
            pltpu.sync_copy(rows0, out_hbm.at[pl.ds(base + j * BT, BT)])
            nxt = jnp.minimum(j + 2, NB - 1)
            pltpu.async_copy(table_hbm.at[idx_v.at[nxt]], rows0, sem0)
            pltpu.make_async_copy(table_hbm.at[idx_v.at[j]], rows1, sem1).wait()
            pltpu.sync_copy(rows1, out_hbm.at[pl.ds(base + (j + 1) * BT, BT)])
            return carry

        lax.fori_loop(0, NB // 2, body, 0)
        pltpu.make_async_copy(table_hbm.at[idx_v.at[0]], rows0, sem0).wait()

    return k(table, idx3)


def _sc_scatter(vals, src3, dst3, zeros):
    NW, NB, BT = src3.shape
    E, D = vals.shape
    N = zeros.shape[0]
    RPW = N // _NS
    mesh = plsc.VectorSubcoreMesh(core_axis_name="c", subcore_axis_name="s")

    @functools.partial(
        pl.kernel,
        out_type=jax.ShapeDtypeStruct((_NC, N, D), jnp.float32),
        mesh=mesh,
        scratch_types=[
            pltpu.VMEM((NB, BT), jnp.int32),
            pltpu.VMEM((NB, BT), jnp.int32),
            pltpu.VMEM((BT, D), jnp.float32),
            pltpu.VMEM((BT, D), jnp.float32),
            pltpu.SemaphoreType.DMA,
            pltpu.SemaphoreType.DMA,
            pltpu.VMEM_SHARED((N, D), jnp.float32),
        ],
    )
    def k(vals_hbm, src_hbm, dst_hbm, zeros_hbm, out_hbm, idxs_v, idxd_v,
          rows0, rows1, sem0, sem1, acc):
        c = lax.axis_index("c")
        s = lax.axis_index("s")
        wid = s * _NC + c
        pltpu.sync_copy(zeros_hbm.at[pl.ds(s * RPW, RPW)],
                        acc.at[pl.ds(s * RPW, RPW)])
        pltpu.sync_copy(src_hbm.at[wid], idxs_v)
        pltpu.sync_copy(dst_hbm.at[wid], idxd_v)
        plsc.subcore_barrier()
        base = wid * (NB * BT)

        def ld(j, buf, sem):
            pltpu.async_copy(vals_hbm.at[pl.ds(base + j * BT, BT)], buf, sem)

        def wt(j, buf, sem):
            pltpu.make_async_copy(vals_hbm.at[pl.ds(base + j * BT, BT)],
                                  buf, sem).wait()

        ld(0, rows0, sem0)

        def body(j2, carry):
            j = j2 * 2
            ld(j + 1, rows1, sem1)
            wt(j, rows0, sem0)
            pltpu.sync_copy(rows0, acc.at[idxs_v.at[j]], add=True)
            pltpu.sync_copy(rows0, acc.at[idxd_v.at[j]], add=True)
            ld(jnp.minimum(j + 2, NB - 1), rows0, sem0)
            wt(j + 1, rows1, sem1)
            pltpu.sync_copy(rows1, acc.at[idxs_v.at[j + 1]], add=True)
            pltpu.sync_copy(rows1, acc.at[idxd_v.at[j + 1]], add=True)
            return carry

        lax.fori_loop(0, NB // 2, body, 0)
        wt(0, rows0, sem0)
        plsc.subcore_barrier()
        pltpu.sync_copy(acc.at[pl.ds(s * RPW, RPW)],
                        out_hbm.at[c, pl.ds(s * RPW, RPW)])

    return k(vals, src3, dst3, zeros)


def _mlp_body(gs_ref, gd_ref, w1_ref, b1_ref, w2_ref, b2_ref, h1e_ref,
              u_ref, ersum_ref):
    gs = gs_ref[...]
    gd = gd_ref[...]
    w1a = w1_ref[0:64, :]
    w1b = w1_ref[64:128, :]
    b1 = b1_ref[...]
    w2 = w2_ref[...]
    b2 = b2_ref[...]
    h1e = h1e_ref[...]

    def seedpart(cs):
        h = jnp.maximum(
            jnp.dot(gs[:, cs:cs + 64], w1a, preferred_element_type=jnp.float32)
            + jnp.dot(gd[:, cs:cs + 64], w1b, preferred_element_type=jnp.float32)
            + b1, 0.0)
        return jnp.dot(h, w2, preferred_element_type=jnp.float32) + b2

    er0 = seedpart(0)
    er1 = seedpart(64)
    u0 = jnp.dot(er0, h1e, preferred_element_type=jnp.float32)
    u1 = jnp.dot(er1, h1e, preferred_element_type=jnp.float32)
    u_ref[...] = jnp.concatenate([u0, u1], axis=1)

    @pl.when(pl.program_id(0) == 0)
    def _():
        ersum_ref[...] = jnp.zeros_like(ersum_ref)

    ersum_ref[...] += jnp.sum(er0 + er1, axis=0, keepdims=True)


def _tc_mlp(G, W1, b1, W2, b2, H1e, E):
    BLK = 2048
    nblk = E // BLK
    return pl.pallas_call(
        _mlp_body,
        grid=(nblk,),
        in_specs=[
            pl.BlockSpec((BLK, 128), lambda i: (i, 0)),
            pl.BlockSpec((BLK, 128), lambda i, nblk=nblk: (i + nblk, 0)),
            pl.BlockSpec((128, 128), lambda i: (0, 0)),
            pl.BlockSpec((1, 128), lambda i: (0, 0)),
            pl.BlockSpec((128, 64), lambda i: (0, 0)),
            pl.BlockSpec((1, 64), lambda i: (0, 0)),
            pl.BlockSpec((64, 64), lambda i: (0, 0)),
        ],
        out_specs=[
            pl.BlockSpec((BLK, 128), lambda i: (i, 0)),
            pl.BlockSpec((1, 64), lambda i: (0, 0)),
        ],
        out_shape=[
            jax.ShapeDtypeStruct((E, 128), jnp.float32),
            jax.ShapeDtypeStruct((1, 64), jnp.float32),
        ],
    )(G, G, W1, b1.reshape(1, 128), W2, b2.reshape(1, 64), H1e)


def _conv2_body(gs_ref, gd_ref, h1b_ref, h2w_ref, z0_ref, z1_ref):
    gs = gs_ref[...]
    gd = gd_ref[...]
    h1b = h1b_ref[...]
    h2w = h2w_ref[...]
    zs = []
    for cs in (0, 64):
        h = jnp.tanh(0.5 * (gs[:, cs:cs + 64] + gd[:, cs:cs + 64]) + h1b)
        zs.append(jnp.dot(h, h2w, preferred_element_type=jnp.float32))
    z0_ref[...] = zs[0][:, 0]
    z1_ref[...] = zs[1][:, 0]


def _tc_conv2(G2, H1b, H2w, E):
    BLK = 2048
    nblk = E // BLK
    return pl.pallas_call(
        _conv2_body,
        grid=(nblk,),
        in_specs=[
            pl.BlockSpec((BLK, 128), lambda i: (i, 0)),
            pl.BlockSpec((BLK, 128), lambda i, nblk=nblk: (i + nblk, 0)),
            pl.BlockSpec((1, 64), lambda i: (0, 0)),
            pl.BlockSpec((64, 1), lambda i: (0, 0)),
        ],
        out_specs=[
            pl.BlockSpec((BLK,), lambda i: (i,)),
            pl.BlockSpec((BLK,), lambda i: (i,)),
        ],
        out_shape=[
            jax.ShapeDtypeStruct((E,), jnp.float32),
            jax.ShapeDtypeStruct((E,), jnp.float32),
        ],
    )(G2, G2, H1b.reshape(1, 64), H2w)


def _sc_scatter_scalar2(z0, z1, src_f, dst_f, N, zeros2):
    E = z0.shape[0]
    EPW = E // _NW
    NG = EPW // 16
    NL = N * 16
    mesh = plsc.VectorSubcoreMesh(core_axis_name="c", subcore_axis_name="s")

    @functools.partial(
        pl.kernel,
        out_type=jax.ShapeDtypeStruct((_NW, 2, NL), jnp.float32),
        mesh=mesh,
        compiler_params=pltpu.CompilerParams(needs_layout_passes=False),
        scratch_types=[
            pltpu.VMEM((EPW,), jnp.int32),
            pltpu.VMEM((EPW,), jnp.int32),
            pltpu.VMEM((EPW,), jnp.float32),
            pltpu.VMEM((EPW,), jnp.float32),
            pltpu.VMEM((NL,), jnp.float32),
            pltpu.VMEM((NL,), jnp.float32),
        ],
    )
    def k(z0_hbm, z1_hbm, src_hbm, dst_hbm, zeros_hbm, out_hbm,
          s_v, d_v, z0_v, z1_v, acc0, acc1):
        c = lax.axis_index("c")
        s = lax.axis_index("s")
        wid = s * _NC + c
        base = wid * EPW
        pltpu.sync_copy(src_hbm.at[pl.ds(base, EPW)], s_v)
        pltpu.sync_copy(dst_hbm.at[pl.ds(base, EPW)], d_v)
        pltpu.sync_copy(z0_hbm.at[pl.ds(base, EPW)], z0_v)
        pltpu.sync_copy(z1_hbm.at[pl.ds(base, EPW)], z1_v)
        pltpu.sync_copy(zeros_hbm, acc0)
        pltpu.sync_copy(zeros_hbm, acc1)
        lanes = lax.iota(jnp.int32, 16)

        def body(g, carry):
            sv = s_v[pl.ds(g * 16, 16)] * 16 + lanes
            dv = d_v[pl.ds(g * 16, 16)] * 16 + lanes
            z0g = z0_v[pl.ds(g * 16, 16)]
            z1g = z1_v[pl.ds(g * 16, 16)]
            plsc.addupdate_scatter(acc0, [sv], z0g)
            plsc.addupdate_scatter(acc0, [dv], z0g)
            plsc.addupdate_scatter(acc1, [sv], z1g)
            plsc.addupdate_scatter(acc1, [dv], z1g)
            return carry

        lax.fori_loop(0, NG, body, 0)
        pltpu.sync_copy(acc0, out_hbm.at[wid, 0])
        pltpu.sync_copy(acc1, out_hbm.at[wid, 1])

    return k(z0, z1, src_f, dst_f, zeros2)


def _sc_finale(wtab_flat, src_f, dst_f, h2b16):
    E = src_f.shape[0]
    EPW = E // _NW
    NG = EPW // 16
    TN = wtab_flat.shape[0]
    mesh = plsc.VectorSubcoreMesh(core_axis_name="c", subcore_axis_name="s")

    @functools.partial(
        pl.kernel,
        out_type=jax.ShapeDtypeStruct((E,), jnp.float32),
        mesh=mesh,
        compiler_params=pltpu.CompilerParams(needs_layout_passes=False),
        scratch_types=[
            pltpu.VMEM((TN,), jnp.float32),
            pltpu.VMEM((EPW,), jnp.int32),
            pltpu.VMEM((EPW,), jnp.int32),
            pltpu.VMEM((EPW,), jnp.float32),
            pltpu.VMEM((16,), jnp.float32),
        ],
    )
    def k(wt_hbm, src_hbm, dst_hbm, h2b_hbm, out_hbm, wt_v, s_v, d_v, o_v,
          h2b_v):
        c = lax.axis_index("c")
        s = lax.axis_index("s")
        wid = s * _NC + c
        base = wid * EPW
        pltpu.sync_copy(wt_hbm, wt_v)
        pltpu.sync_copy(h2b_hbm, h2b_v)
        pltpu.sync_copy(src_hbm.at[pl.ds(base, EPW)], s_v)
        pltpu.sync_copy(dst_hbm.at[pl.ds(base, EPW)], d_v)

        def body(g, carry):
            sv = s_v[pl.ds(g * 16, 16)] * 2
            dv = d_v[pl.ds(g * 16, 16)] * 2
            h2b = h2b_v[...]
            a0 = plsc.load_gather(wt_v, [sv])
            b0 = plsc.load_gather(wt_v, [dv])
            a1 = plsc.load_gather(wt_v, [sv + 1])
            b1 = plsc.load_gather(wt_v, [dv + 1])
            q0 = 0.5 * (a0 + b0) + h2b
            q1 = 0.5 * (a1 + b1) + h2b
            w0 = 1.0 / (1.0 + jnp.exp(-q0))
            w1 = 1.0 / (1.0 + jnp.exp(-q1))
            o_v[pl.ds(g * 16, 16)] = 0.5 * (w0 + w1)
            return carry

        lax.fori_loop(0, NG, body, 0)
        pltpu.sync_copy(o_v, out_hbm.at[pl.ds(base, EPW)])

    return k(wtab_flat, src_f, dst_f, h2b16)


def kernel(x, edge_index, batch, t, W1, b1, W2, b2, H1w, H1b, H2w, H2b):
    N = x.shape[0]
    E = edge_index.shape[1]
    src, dst = edge_index[0], edge_index[1]

    adj = jnp.zeros((N, N), dtype=jnp.float32).at[src, dst].add(1.0)

    H1e, H1x = H1w[:64], H1w[64:]
    K = _SVD_DIM + 10
    Oms = []
    for s in _SVD_SEEDS:
        Om = jax.random.normal(jax.random.key(s), (N, K), dtype=jnp.float32)
        Oms.append(jnp.pad(Om, ((0, 0), (0, 128 - K))))
    Y0p, Y1p, aux = _tc_power_aux(adj, Oms[0], Oms[1], x, H1x)
    xterm = aux[:, :64]
    rowd, cold = aux[:, 64], aux[:, 65]
    Bdeg = rowd + cold
    Binv = jnp.where(Bdeg > 0, 1.0 / Bdeg, 0.0)

    Qs = [jnp.linalg.qr(Yp[:, :K])[0] for Yp in (Y0p, Y1p)]
    B0, B1 = _tc_qta(jnp.pad(Qs[0], ((0, 0), (0, 128 - K))),
                     jnp.pad(Qs[1], ((0, 0), (0, 128 - K))), adj)
    emb = []
    for Q, Bp in zip(Qs, (B0, B1)):
        Ub, sv, _ = jnp.linalg.svd(Bp[:K], full_matrices=False)
        emb.append(jax.lax.stop_gradient(
            Q @ (Ub[:, :_SVD_DIM] * sv[:_SVD_DIM])))

    Temb = jnp.concatenate(emb, axis=1)
    idx_all = jnp.concatenate([src, dst]).reshape(_NW, -1, 128)
    src3 = src.reshape(_NW, -1, 128)
    dst3 = dst.reshape(_NW, -1, 128)

    Gemb = _sc_gather(Temb, idx_all)
    U, ersum = _tc_mlp(Gemb, W1, b1, W2, b2, H1e, E)
    edge_pool = ersum / (2.0 * E)

    acc2sc = _sc_scatter(U, src3, dst3, jnp.zeros((N, 128), jnp.float32))
    e1 = Binv[:, None] * (acc2sc[0] + acc2sc[1]
                          + jnp.concatenate([xterm, xterm], axis=1))

    G2 = _sc_gather(e1, idx_all)
    z0, z1 = _tc_conv2(G2, H1b, H2w, E)
    zacc = _sc_scatter_scalar2(z0, z1, src, dst, N,
                               jnp.zeros((N * 16,), jnp.float32))
    zsum = zacc.sum(axis=0).reshape(2, N, 16).sum(axis=2)
    wtab_flat = (zsum * Binv[None, :]).T.reshape(-1)

    weights = _sc_finale(wtab_flat, src, dst,
                         jnp.full((16,), H2b[0], jnp.float32))
    return weights, edge_pool
